# baseline (speedup 1.0000x reference)
import jax
import jax.numpy as jnp
from jax import lax
from jax.experimental import pallas as pl
from jax.experimental.pallas import tpu as pltpu
from jax.experimental.pallas import tpu_sc as plsc

DIM = 32
TOTAL = 1_638_400
LANES = 128
NC, NS = 2, 16
NW = NC * NS
TOK_PER_W = TOTAL // NW
G = 4
CH = G * LANES
N_CHUNKS = TOK_PER_W // CH
NBUF = 2
SUB = 8
TGRP = DIM // SUB
NBLK = TOTAL // LANES


def _body(table_hbm, idx_hbm, o4_hbm, idx_v, rows_v, t4_v, gsems, ssems):
    wid = lax.axis_index("s") * NC + lax.axis_index("c")
    tok0 = wid * TOK_PER_W
    blk0 = wid * (TOK_PER_W // LANES)
    lane16 = lax.iota(jnp.int32, 16)

    def fire(b, c):
        base = tok0 + c * CH
        pltpu.sync_copy(idx_hbm.at[pl.ds(base, CH)], idx_v.at[b])
        for g in range(G):
            pltpu.async_copy(
                table_hbm.at[idx_v.at[b].at[pl.ds(g * LANES, LANES)]],
                rows_v.at[b].at[pl.ds(g * LANES, LANES)],
                gsems.at[b],
            )

    def drain_gathers(b):
        pltpu.make_async_copy(
            o4_hbm.at[0].at[pl.ds(0, G)], rows_v.at[b], gsems.at[b]
        ).wait()

    def wait_stores(b):
        pltpu.make_async_copy(
            t4_v.at[b], o4_hbm.at[0].at[pl.ds(0, G)], ssems.at[b]
        ).wait()

    def transpose(b):
        def per_q(q, carry):
            tc = q // G
            g = q % G
            for s in range(SUB):
                feat = jnp.broadcast_to(tc * SUB + s, (16,)).astype(jnp.int32)
                for m in range(LANES // 16):
                    tok = lane16 + (g * LANES + m * 16)
                    vec = plsc.load_gather(rows_v.at[b], [tok, feat])
                    t4_v[b, tc, g, s, pl.ds(m * 16, 16)] = vec
            return carry

        lax.fori_loop(0, TGRP * G, per_q, 0)

    def store(b, c):
        for tc in range(TGRP):
            pltpu.async_copy(
                t4_v.at[b].at[tc],
                o4_hbm.at[tc].at[pl.ds(blk0 + c * G, G)],
                ssems.at[b],
            )

    for b in range(NBUF):
        fire(b, b)

    def outer(i, carry):
        c0 = i * NBUF
        for b in range(NBUF):
            c = c0 + b
            drain_gathers(b)

            @pl.when(c >= NBUF)
            def _():
                wait_stores(b)

            transpose(b)
            store(b, c)

            @pl.when(c + NBUF < N_CHUNKS)
            def _():
                fire(b, c + NBUF)

        return carry

    lax.fori_loop(0, N_CHUNKS // NBUF, outer, 0)
    for b in range(NBUF):
        wait_stores(b)


def kernel(x_data, table):
    idx = x_data.astype(jnp.int32)
    mesh = plsc.VectorSubcoreMesh(core_axis_name="c", subcore_axis_name="s")
    f = pl.kernel(
        _body,
        mesh=mesh,
        out_type=jax.ShapeDtypeStruct((TGRP, NBLK, SUB, LANES), jnp.float32),
        scratch_types=[
            pltpu.VMEM((NBUF, CH), jnp.int32),
            pltpu.VMEM((NBUF, CH, DIM), jnp.float32),
            pltpu.VMEM((NBUF, TGRP, G, SUB, LANES), jnp.float32),
            pltpu.SemaphoreType.DMA((NBUF,)),
            pltpu.SemaphoreType.DMA((NBUF,)),
        ],
        compiler_params=pltpu.CompilerParams(
            use_tc_tiling_on_sc=False, needs_layout_passes=False
        ),
    )
    o4 = f(table, idx)
    return o4.transpose(1, 3, 0, 2).reshape(TOTAL, DIM)


# speedup vs baseline: 1.7964x; 1.7964x over previous
import jax
import jax.numpy as jnp
from jax import lax
from jax.experimental import pallas as pl
from jax.experimental.pallas import tpu as pltpu
from jax.experimental.pallas import tpu_sc as plsc

DIM = 32
TOTAL = 1_638_400
LANES = 128
NC, NS = 2, 16
NW = NC * NS
TOK_PER_W = TOTAL // NW
G = 4
CH = G * LANES
N_CHUNKS = TOK_PER_W // CH
NBUF = 2
SUB = 8
TGRP = DIM // SUB
NBLK = TOTAL // LANES
TILE_W = G * SUB * LANES
T4 = TGRP * TILE_W


def _body(table_hbm, idx_hbm, o4_hbm, idx_v, rows_v, t4_v, ftab, ptab, gsems, ssems):
    wid = lax.axis_index("s") * NC + lax.axis_index("c")
    tok0 = wid * TOK_PER_W
    blk0 = wid * (TOK_PER_W // LANES)
    lane16 = lax.iota(jnp.int32, 16)

    for f in range(DIM):
        fv = lax.rem(lane16 + f, DIM)
        ftab[f, :] = fv
        ptab[f, :] = (fv // SUB) * TILE_W + lax.rem(fv, SUB) * LANES + lane16

    def fire(b, c):
        base = tok0 + c * CH
        pltpu.sync_copy(idx_hbm.at[pl.ds(base, CH)], idx_v.at[b])
        for g in range(G):
            pltpu.async_copy(
                table_hbm.at[idx_v.at[b].at[pl.ds(g * LANES, LANES)]],
                rows_v.at[b].at[pl.ds(g * LANES, LANES)],
                gsems.at[b],
            )

    def drain_gathers(b):
        pltpu.make_async_copy(
            table_hbm.at[pl.ds(0, CH)], rows_v.at[b], gsems.at[b]
        ).wait()

    def wait_stores(b):
        pltpu.make_async_copy(
            o4_hbm.at[0].at[pl.ds(0, T4)], t4_v.at[b], ssems.at[b]
        ).wait()

    def transpose(b):
        def per_f(f, carry):
            fv = ftab[f, :]
            pv = ptab[f, :]
            for g in range(G):
                for m in range(LANES // 16):
                    tok = lane16 + (g * LANES + m * 16)
                    vec = plsc.load_gather(rows_v.at[b], [tok, fv])
                    plsc.store_scatter(
                        t4_v.at[b], [pv + (g * SUB * LANES + m * 16)], vec
                    )
            return carry

        lax.fori_loop(0, DIM, per_f, 0)

    def store(b, c):
        for tc in range(TGRP):
            pltpu.async_copy(
                t4_v.at[b].at[pl.ds(tc * TILE_W, TILE_W)],
                o4_hbm.at[tc].at[pl.ds((blk0 + c * G) * SUB * LANES, TILE_W)],
                ssems.at[b],
            )

    for b in range(NBUF):
        fire(b, b)

    def outer(i, carry):
        c0 = i * NBUF
        for b in range(NBUF):
            c = c0 + b
            drain_gathers(b)

            @pl.when(c >= NBUF)
            def _():
                wait_stores(b)

            transpose(b)
            store(b, c)

            @pl.when(c + NBUF < N_CHUNKS)
            def _():
                fire(b, c + NBUF)

        return carry

    lax.fori_loop(0, N_CHUNKS // NBUF, outer, 0)
    for b in range(NBUF):
        wait_stores(b)


def kernel(x_data, table):
    idx = x_data.astype(jnp.int32)
    mesh = plsc.VectorSubcoreMesh(core_axis_name="c", subcore_axis_name="s")
    f = pl.kernel(
        _body,
        mesh=mesh,
        out_type=jax.ShapeDtypeStruct((TGRP, NBLK * SUB * LANES), jnp.float32),
        scratch_types=[
            pltpu.VMEM((NBUF, CH), jnp.int32),
            pltpu.VMEM((NBUF, CH, DIM), jnp.float32),
            pltpu.VMEM((NBUF, T4), jnp.float32),
            pltpu.VMEM((DIM, 16), jnp.int32),
            pltpu.VMEM((DIM, 16), jnp.int32),
            pltpu.SemaphoreType.DMA((NBUF,)),
            pltpu.SemaphoreType.DMA((NBUF,)),
        ],
        compiler_params=pltpu.CompilerParams(
            use_tc_tiling_on_sc=False, needs_layout_passes=False
        ),
    )
    o4 = f(table, idx)
    return (
        o4.reshape(TGRP, NBLK, SUB, LANES)
        .transpose(1, 3, 0, 2)
        .reshape(TOTAL, DIM)
    )


# speedup vs baseline: 1.9217x; 1.0697x over previous
import jax
import jax.numpy as jnp
from jax import lax
from jax.experimental import pallas as pl
from jax.experimental.pallas import tpu as pltpu
from jax.experimental.pallas import tpu_sc as plsc

DIM = 32
TOTAL = 1_638_400
LANES = 128
NC, NS = 2, 16
NW = NC * NS
TOK_PER_W = TOTAL // NW
G = 4
CH = G * LANES
N_CHUNKS = TOK_PER_W // CH
NBUF = 2
SUB = 8
TGRP = DIM // SUB
NBLK = TOTAL // LANES
TILE_W = G * SUB * LANES
T4 = TGRP * TILE_W


def _body(
    table_hbm, idx_hbm, o4_hbm, idx_v, rows_v, t4_v, ftab, ptab, gsems, ssems, isems
):
    wid = lax.axis_index("s") * NC + lax.axis_index("c")
    tok0 = wid * TOK_PER_W
    blk0 = wid * (TOK_PER_W // LANES)
    lane16 = lax.iota(jnp.int32, 16)

    for f in range(DIM):
        fv = lax.rem(lane16 + f, DIM)
        ftab[f, :] = fv
        ptab[f, :] = (fv // SUB) * TILE_W + lax.rem(fv, SUB) * LANES + lane16

    def prefetch_idx(b, c):
        base = tok0 + c * CH
        pltpu.async_copy(idx_hbm.at[pl.ds(base, CH)], idx_v.at[b], isems.at[b])

    def fire(b):
        pltpu.make_async_copy(
            idx_hbm.at[pl.ds(0, CH)], idx_v.at[b], isems.at[b]
        ).wait()
        for g in range(G):
            pltpu.async_copy(
                table_hbm.at[idx_v.at[b].at[pl.ds(g * LANES, LANES)]],
                rows_v.at[b].at[pl.ds(g * LANES, LANES)],
                gsems.at[b],
            )

    def drain_gathers(b):
        pltpu.make_async_copy(
            table_hbm.at[pl.ds(0, CH)], rows_v.at[b], gsems.at[b]
        ).wait()

    def wait_stores(b):
        pltpu.make_async_copy(
            o4_hbm.at[0].at[pl.ds(0, T4)], t4_v.at[b], ssems.at[b]
        ).wait()

    def transpose(b):
        def per_f(f, carry):
            fv = ftab[f, :]
            pv = ptab[f, :]
            for g in range(G):
                for m in range(LANES // 16):
                    tok = lane16 + (g * LANES + m * 16)
                    vec = plsc.load_gather(rows_v.at[b], [tok, fv])
                    plsc.store_scatter(
                        t4_v.at[b], [pv + (g * SUB * LANES + m * 16)], vec
                    )
            return carry

        lax.fori_loop(0, DIM, per_f, 0)

    def store(b, c):
        for tc in range(TGRP):
            pltpu.async_copy(
                t4_v.at[b].at[pl.ds(tc * TILE_W, TILE_W)],
                o4_hbm.at[tc].at[pl.ds((blk0 + c * G) * SUB * LANES, TILE_W)],
                ssems.at[b],
            )

    for b in range(NBUF):
        prefetch_idx(b, b)
    for b in range(NBUF):
        fire(b)

    def outer(i, carry):
        c0 = i * NBUF
        for b in range(NBUF):
            c = c0 + b
            drain_gathers(b)

            @pl.when(c + NBUF < N_CHUNKS)
            def _():
                prefetch_idx(b, c + NBUF)

            @pl.when(c >= NBUF)
            def _():
                wait_stores(b)

            transpose(b)
            store(b, c)

            @pl.when(c + NBUF < N_CHUNKS)
            def _():
                fire(b)

        return carry

    lax.fori_loop(0, N_CHUNKS // NBUF, outer, 0)
    for b in range(NBUF):
        wait_stores(b)


def kernel(x_data, table):
    idx = x_data.astype(jnp.int32)
    mesh = plsc.VectorSubcoreMesh(core_axis_name="c", subcore_axis_name="s")
    f = pl.kernel(
        _body,
        mesh=mesh,
        out_type=jax.ShapeDtypeStruct((TGRP, NBLK * SUB * LANES), jnp.float32),
        scratch_types=[
            pltpu.VMEM((NBUF, CH), jnp.int32),
            pltpu.VMEM((NBUF, CH, DIM), jnp.float32),
            pltpu.VMEM((NBUF, T4), jnp.float32),
            pltpu.VMEM((DIM, 16), jnp.int32),
            pltpu.VMEM((DIM, 16), jnp.int32),
            pltpu.SemaphoreType.DMA((NBUF,)),
            pltpu.SemaphoreType.DMA((NBUF,)),
            pltpu.SemaphoreType.DMA((NBUF,)),
        ],
        compiler_params=pltpu.CompilerParams(
            use_tc_tiling_on_sc=False, needs_layout_passes=False
        ),
    )
    o4 = f(table, idx)
    return (
        o4.reshape(TGRP, NBLK, SUB, LANES)
        .transpose(1, 3, 0, 2)
        .reshape(TOTAL, DIM)
    )


# speedup vs baseline: 2.6864x; 1.3979x over previous
import jax
import jax.numpy as jnp
from jax import lax
from jax.experimental import pallas as pl
from jax.experimental.pallas import tpu as pltpu
from jax.experimental.pallas import tpu_sc as plsc

DIM = 32
TOTAL = 1_638_400
LANES = 128
NC, NS = 2, 16
NW = NC * NS
TOK_PER_W = TOTAL // NW
G = 4
CH = G * LANES
N_CHUNKS = TOK_PER_W // CH
NBUF = 2
SUB = 8
TGRP = DIM // SUB
NBLK = TOTAL // LANES
TILE_W = G * SUB * LANES
T4 = TGRP * TILE_W


def _body(
    table_hbm, idx_hbm, o4_hbm, idx_v, rows_v, t4_v, ftab, ptab, gsems, ssems, isems
):
    wid = lax.axis_index("s") * NC + lax.axis_index("c")
    tok0 = wid * TOK_PER_W
    blk0 = wid * (TOK_PER_W // LANES)
    lane16 = lax.iota(jnp.int32, 16)

    for f in range(DIM):
        fv = lax.rem(lane16 + f, DIM)
        ftab[f, :] = fv
        ptab[f, :] = (fv // SUB) * TILE_W + lax.rem(fv, SUB) * LANES + lane16

    def prefetch_idx(b, c):
        base = tok0 + c * CH
        pltpu.async_copy(idx_hbm.at[pl.ds(base, CH)], idx_v.at[b], isems.at[b])

    def fire(b):
        pltpu.make_async_copy(
            idx_hbm.at[pl.ds(0, CH)], idx_v.at[b], isems.at[b]
        ).wait()
        for g in range(G):
            pltpu.async_copy(
                table_hbm.at[idx_v.at[b].at[pl.ds(g * LANES, LANES)]],
                rows_v.at[b].at[pl.ds(g * LANES, LANES)],
                gsems.at[b],
            )

    def drain_gathers(b):
        pltpu.make_async_copy(
            table_hbm.at[pl.ds(0, CH)], rows_v.at[b], gsems.at[b]
        ).wait()

    def wait_stores(b):
        pltpu.make_async_copy(
            o4_hbm.at[0].at[pl.ds(0, T4)], t4_v.at[b], ssems.at[b]
        ).wait()

    def transpose(b):
        @plsc.parallel_loop(0, DIM)
        def per_f(f):
            fv = ftab[f, :]
            pv = ptab[f, :]
            for g in range(G):
                for m in range(LANES // 16):
                    tok = lane16 + (g * LANES + m * 16)
                    vec = plsc.load_gather(rows_v.at[b], [tok, fv])
                    plsc.store_scatter(
                        t4_v.at[b], [pv + (g * SUB * LANES + m * 16)], vec
                    )

    def store(b, c):
        for tc in range(TGRP):
            pltpu.async_copy(
                t4_v.at[b].at[pl.ds(tc * TILE_W, TILE_W)],
                o4_hbm.at[tc].at[pl.ds((blk0 + c * G) * SUB * LANES, TILE_W)],
                ssems.at[b],
            )

    for b in range(NBUF):
        prefetch_idx(b, b)
    for b in range(NBUF):
        fire(b)

    def outer(i, carry):
        c0 = i * NBUF
        for b in range(NBUF):
            c = c0 + b
            drain_gathers(b)

            @pl.when(c + NBUF < N_CHUNKS)
            def _():
                prefetch_idx(b, c + NBUF)

            @pl.when(c >= NBUF)
            def _():
                wait_stores(b)

            transpose(b)
            store(b, c)

            @pl.when(c + NBUF < N_CHUNKS)
            def _():
                fire(b)

        return carry

    lax.fori_loop(0, N_CHUNKS // NBUF, outer, 0)
    for b in range(NBUF):
        wait_stores(b)


def kernel(x_data, table):
    idx = x_data.astype(jnp.int32)
    mesh = plsc.VectorSubcoreMesh(core_axis_name="c", subcore_axis_name="s")
    f = pl.kernel(
        _body,
        mesh=mesh,
        out_type=jax.ShapeDtypeStruct((TGRP, NBLK * SUB * LANES), jnp.float32),
        scratch_types=[
            pltpu.VMEM((NBUF, CH), jnp.int32),
            pltpu.VMEM((NBUF, CH, DIM), jnp.float32),
            pltpu.VMEM((NBUF, T4), jnp.float32),
            pltpu.VMEM((DIM, 16), jnp.int32),
            pltpu.VMEM((DIM, 16), jnp.int32),
            pltpu.SemaphoreType.DMA((NBUF,)),
            pltpu.SemaphoreType.DMA((NBUF,)),
            pltpu.SemaphoreType.DMA((NBUF,)),
        ],
        compiler_params=pltpu.CompilerParams(
            use_tc_tiling_on_sc=False, needs_layout_passes=False
        ),
    )
    o4 = f(table, idx)
    return (
        o4.reshape(TGRP, NBLK, SUB, LANES)
        .transpose(1, 3, 0, 2)
        .reshape(TOTAL, DIM)
    )


# speedup vs baseline: 5.2848x; 1.9672x over previous
import jax
import jax.numpy as jnp
from jax import lax
from jax.experimental import pallas as pl
from jax.experimental.pallas import tpu as pltpu
from jax.experimental.pallas import tpu_sc as plsc

DIM = 32
TOTAL = 1_638_400
LANES = 128
NC, NS = 2, 16
NW = NC * NS
TOK_PER_W = TOTAL // NW
G = 4
CH = G * LANES
N_CHUNKS = TOK_PER_W // CH
NBUF = 2
SUB = 8
TGRP = DIM // SUB
NBLK = TOTAL // LANES
TILE_W = G * SUB * LANES
T4 = TGRP * TILE_W


def _body(
    table_hbm, idx_hbm, o4_hbm, idx_v, rows_v, t4_v, ftab, ptab, gsems, ssems, isems
):
    wid = lax.axis_index("s") * NC + lax.axis_index("c")
    tok0 = wid * TOK_PER_W
    blk0 = wid * (TOK_PER_W // LANES)
    lane16 = lax.iota(jnp.int32, 16)

    for f in range(DIM):
        fv = lax.rem(lane16 + f, DIM)
        ftab[f, :] = fv
        ptab[f, :] = (fv // SUB) * TILE_W + lax.rem(fv, SUB) * LANES + lane16

    def prefetch_idx(b, c):
        base = tok0 + c * CH
        pltpu.async_copy(idx_hbm.at[pl.ds(base, CH)], idx_v.at[b], isems.at[b])

    def fire(b):
        pltpu.make_async_copy(
            idx_hbm.at[pl.ds(0, CH)], idx_v.at[b], isems.at[b]
        ).wait()
        for g in range(G):
            pltpu.async_copy(
                table_hbm.at[idx_v.at[b].at[pl.ds(g * LANES, LANES)]],
                rows_v.at[b].at[pl.ds(g * LANES, LANES)],
                gsems.at[b],
            )

    def drain_gathers(b):
        pltpu.make_async_copy(
            table_hbm.at[pl.ds(0, CH)], rows_v.at[b], gsems.at[b]
        ).wait()

    def wait_stores(b):
        pltpu.make_async_copy(
            o4_hbm.at[0].at[pl.ds(0, T4)], t4_v.at[b], ssems.at[b]
        ).wait()

    def transpose(b):
        @plsc.parallel_loop(0, DIM)
        def per_f(f):
            fv = ftab[f, :]
            pv = ptab[f, :]
            for g in range(G):
                for m in range(LANES // 16):
                    tok = lane16 + (g * LANES + m * 16)
                    vec = plsc.load_gather(rows_v.at[b], [tok, fv])
                    plsc.store_scatter(
                        t4_v.at[b], [pv + (g * SUB * LANES + m * 16)], vec
                    )

    def store(b, c):
        for tc in range(TGRP):
            pltpu.async_copy(
                t4_v.at[b].at[pl.ds(tc * TILE_W, TILE_W)],
                o4_hbm.at[tc].at[pl.ds((blk0 + c * G) * SUB * LANES, TILE_W)],
                ssems.at[b],
            )

    for b in range(NBUF):
        prefetch_idx(b, b)
    for b in range(NBUF):
        fire(b)

    def outer(i, carry):
        c0 = i * NBUF
        for b in range(NBUF):
            c = c0 + b
            drain_gathers(b)

            @pl.when(c + NBUF < N_CHUNKS)
            def _():
                prefetch_idx(b, c + NBUF)

            @pl.when(c >= NBUF)
            def _():
                wait_stores(b)

            transpose(b)
            store(b, c)

            @pl.when(c + NBUF < N_CHUNKS)
            def _():
                fire(b)

        return carry

    lax.fori_loop(0, N_CHUNKS // NBUF, outer, 0)
    for b in range(NBUF):
        wait_stores(b)


V = 1_000_000
NT = V // LANES
TAIL = V - NT * LANES
W1 = NT // NW
NBUF1 = 2


def _lin_body(
    tabT_hbm, tail_hbm, lin_hbm, buf0, buf1, bufT0, bufT1, tbuf, f1tab, p1tab,
    wsems, ssems,
):
    wid = lax.axis_index("s") * NC + lax.axis_index("c")
    lane16 = lax.iota(jnp.int32, 16)
    bufs = (buf0, buf1)
    bufTs = (bufT0, bufT1)

    for f in range(DIM):
        fv = lax.rem(lane16 + f, DIM)
        f1tab[f, :] = fv
        p1tab[f, :] = lane16 * DIM + fv

    def fetch(b, t):
        pltpu.async_copy(
            tabT_hbm.at[:, pl.ds(t * LANES, LANES)], bufs[b], wsems.at[b]
        )

    def wait_fetch(b):
        pltpu.make_async_copy(
            tabT_hbm.at[:, pl.ds(0, LANES)], bufs[b], wsems.at[b]
        ).wait()

    def transposeT(b):
        @plsc.parallel_loop(0, DIM)
        def per_f(f):
            fv = f1tab[f, :]
            pv = p1tab[f, :]
            for m in range(LANES // 16):
                tok = lane16 + m * 16
                vec = plsc.load_gather(bufs[b], [fv, tok])
                plsc.store_scatter(bufTs[b], [pv + m * 16 * DIM], vec)

    def store_lin(b, t):
        pltpu.async_copy(
            bufTs[b],
            lin_hbm.at[pl.ds(t * LANES * DIM, LANES * DIM)],
            ssems.at[b],
        )

    def wait_store(b):
        pltpu.make_async_copy(
            lin_hbm.at[pl.ds(0, LANES * DIM)], bufTs[b], ssems.at[b]
        ).wait()

    for b in range(NBUF1):
        fetch(b, b * NW + wid)

    def outer(j, carry):
        for b in range(NBUF1):
            t = (j * NBUF1 + b) * NW + wid
            wait_fetch(b)

            @pl.when(j >= 1)
            def _():
                wait_store(b)

            transposeT(b)
            store_lin(b, t)

            @pl.when(j < W1 // NBUF1 - 1)
            def _():
                fetch(b, t + NBUF1 * NW)

        return carry

    lax.fori_loop(0, W1 // NBUF1, outer, 0)
    for b in range(NBUF1):
        wait_store(b)

    @pl.when(wid < NT - W1 * NW)
    def _():
        t = W1 * NW + wid
        pltpu.sync_copy(tabT_hbm.at[:, pl.ds(t * LANES, LANES)], buf0)
        transposeT(0)
        pltpu.sync_copy(bufT0, lin_hbm.at[pl.ds(t * LANES * DIM, LANES * DIM)])

    @pl.when(wid == 4)
    def _():
        pltpu.sync_copy(tail_hbm, tbuf)
        pltpu.sync_copy(tbuf, lin_hbm.at[pl.ds(NT * LANES * DIM, TAIL * DIM)])


def _linearize_table(table):
    mesh = plsc.VectorSubcoreMesh(core_axis_name="c", subcore_axis_name="s")
    tabT = table.T
    tail = table[NT * LANES :].reshape(TAIL * DIM)
    f = pl.kernel(
        _lin_body,
        mesh=mesh,
        out_type=jax.ShapeDtypeStruct((V * DIM,), jnp.float32),
        scratch_types=[
            pltpu.VMEM((DIM, LANES), jnp.float32),
            pltpu.VMEM((DIM, LANES), jnp.float32),
            pltpu.VMEM((LANES * DIM,), jnp.float32),
            pltpu.VMEM((LANES * DIM,), jnp.float32),
            pltpu.VMEM((TAIL * DIM,), jnp.float32),
            pltpu.VMEM((DIM, 16), jnp.int32),
            pltpu.VMEM((DIM, 16), jnp.int32),
            pltpu.SemaphoreType.DMA((NBUF1,)),
            pltpu.SemaphoreType.DMA((NBUF1,)),
        ],
        compiler_params=pltpu.CompilerParams(
            use_tc_tiling_on_sc=True, needs_layout_passes=False
        ),
    )
    return f(tabT, tail).reshape(V, DIM)


def kernel(x_data, table):
    idx = x_data.astype(jnp.int32)
    table = _linearize_table(table)
    mesh = plsc.VectorSubcoreMesh(core_axis_name="c", subcore_axis_name="s")
    f = pl.kernel(
        _body,
        mesh=mesh,
        out_type=jax.ShapeDtypeStruct((TGRP, NBLK * SUB * LANES), jnp.float32),
        scratch_types=[
            pltpu.VMEM((NBUF, CH), jnp.int32),
            pltpu.VMEM((NBUF, CH, DIM), jnp.float32),
            pltpu.VMEM((NBUF, T4), jnp.float32),
            pltpu.VMEM((DIM, 16), jnp.int32),
            pltpu.VMEM((DIM, 16), jnp.int32),
            pltpu.SemaphoreType.DMA((NBUF,)),
            pltpu.SemaphoreType.DMA((NBUF,)),
            pltpu.SemaphoreType.DMA((NBUF,)),
        ],
        compiler_params=pltpu.CompilerParams(
            use_tc_tiling_on_sc=False, needs_layout_passes=False
        ),
    )
    o4 = f(table, idx)
    return (
        o4.reshape(TGRP, NBLK, SUB, LANES)
        .transpose(1, 3, 0, 2)
        .reshape(TOTAL, DIM)
    )


# speedup vs baseline: 5.4502x; 1.0313x over previous
import jax
import jax.numpy as jnp
from jax import lax
from jax.experimental import pallas as pl
from jax.experimental.pallas import tpu as pltpu
from jax.experimental.pallas import tpu_sc as plsc

DIM = 32
TOTAL = 1_638_400
LANES = 128
NC, NS = 2, 16
NW = NC * NS
TOK_PER_W = TOTAL // NW
G = 5
CH = G * LANES
N_CHUNKS = TOK_PER_W // CH
NBUF = 2
SUB = 8
TGRP = DIM // SUB
NBLK = TOTAL // LANES
TILE_W = G * SUB * LANES
T4 = TGRP * TILE_W


def _body(
    table_hbm, idx_hbm, o4_hbm, idx_v, rows_v, t4_v, ftab, ptab, gsems, ssems, isems
):
    wid = lax.axis_index("s") * NC + lax.axis_index("c")
    tok0 = wid * TOK_PER_W
    blk0 = wid * (TOK_PER_W // LANES)
    lane16 = lax.iota(jnp.int32, 16)

    for f in range(DIM):
        fv = lax.rem(lane16 + f, DIM)
        ftab[f, :] = fv
        ptab[f, :] = (fv // SUB) * TILE_W + lax.rem(fv, SUB) * LANES + lane16

    def prefetch_idx(b, c):
        base = tok0 + c * CH
        pltpu.async_copy(idx_hbm.at[pl.ds(base, CH)], idx_v.at[b], isems.at[b])

    def fire(b):
        pltpu.make_async_copy(
            idx_hbm.at[pl.ds(0, CH)], idx_v.at[b], isems.at[b]
        ).wait()
        for g in range(G):
            pltpu.async_copy(
                table_hbm.at[idx_v.at[b].at[pl.ds(g * LANES, LANES)]],
                rows_v.at[b].at[pl.ds(g * LANES, LANES)],
                gsems.at[b],
            )

    def drain_gathers(b):
        pltpu.make_async_copy(
            table_hbm.at[pl.ds(0, CH)], rows_v.at[b], gsems.at[b]
        ).wait()

    def wait_stores(b):
        pltpu.make_async_copy(
            o4_hbm.at[0].at[pl.ds(0, T4)], t4_v.at[b], ssems.at[b]
        ).wait()

    def transpose(b):
        @plsc.parallel_loop(0, DIM)
        def per_f(f):
            fv = ftab[f, :]
            pv = ptab[f, :]
            for g in range(G):
                for m in range(LANES // 16):
                    tok = lane16 + (g * LANES + m * 16)
                    vec = plsc.load_gather(rows_v.at[b], [tok, fv])
                    plsc.store_scatter(
                        t4_v.at[b], [pv + (g * SUB * LANES + m * 16)], vec
                    )

    def store(b, c):
        for tc in range(TGRP):
            pltpu.async_copy(
                t4_v.at[b].at[pl.ds(tc * TILE_W, TILE_W)],
                o4_hbm.at[tc].at[pl.ds((blk0 + c * G) * SUB * LANES, TILE_W)],
                ssems.at[b],
            )

    for b in range(NBUF):
        prefetch_idx(b, b)
    for b in range(NBUF):
        fire(b)

    def outer(i, carry):
        c0 = i * NBUF
        for b in range(NBUF):
            c = c0 + b
            drain_gathers(b)

            @pl.when(c + NBUF < N_CHUNKS)
            def _():
                prefetch_idx(b, c + NBUF)

            @pl.when(c >= NBUF)
            def _():
                wait_stores(b)

            transpose(b)
            store(b, c)

            @pl.when(c + NBUF < N_CHUNKS)
            def _():
                fire(b)

        return carry

    lax.fori_loop(0, N_CHUNKS // NBUF, outer, 0)
    for b in range(NBUF):
        wait_stores(b)


V = 1_000_000
NT = V // LANES
TAIL = V - NT * LANES
W1 = NT // NW
NBUF1 = 2


def _lin_body(
    tabT_hbm, tail_hbm, lin_hbm, buf0, buf1, bufT0, bufT1, tbuf, f1tab, p1tab,
    wsems, ssems,
):
    wid = lax.axis_index("s") * NC + lax.axis_index("c")
    lane16 = lax.iota(jnp.int32, 16)
    bufs = (buf0, buf1)
    bufTs = (bufT0, bufT1)

    for f in range(DIM):
        fv = lax.rem(lane16 + f, DIM)
        f1tab[f, :] = fv
        p1tab[f, :] = lane16 * DIM + fv

    def fetch(b, t):
        pltpu.async_copy(
            tabT_hbm.at[:, pl.ds(t * LANES, LANES)], bufs[b], wsems.at[b]
        )

    def wait_fetch(b):
        pltpu.make_async_copy(
            tabT_hbm.at[:, pl.ds(0, LANES)], bufs[b], wsems.at[b]
        ).wait()

    def transposeT(b):
        @plsc.parallel_loop(0, DIM)
        def per_f(f):
            fv = f1tab[f, :]
            pv = p1tab[f, :]
            for m in range(LANES // 16):
                tok = lane16 + m * 16
                vec = plsc.load_gather(bufs[b], [fv, tok])
                plsc.store_scatter(bufTs[b], [pv + m * 16 * DIM], vec)

    def store_lin(b, t):
        pltpu.async_copy(
            bufTs[b],
            lin_hbm.at[pl.ds(t * LANES * DIM, LANES * DIM)],
            ssems.at[b],
        )

    def wait_store(b):
        pltpu.make_async_copy(
            lin_hbm.at[pl.ds(0, LANES * DIM)], bufTs[b], ssems.at[b]
        ).wait()

    for b in range(NBUF1):
        fetch(b, b * NW + wid)

    def outer(j, carry):
        for b in range(NBUF1):
            t = (j * NBUF1 + b) * NW + wid
            wait_fetch(b)

            @pl.when(j >= 1)
            def _():
                wait_store(b)

            transposeT(b)
            store_lin(b, t)

            @pl.when(j < W1 // NBUF1 - 1)
            def _():
                fetch(b, t + NBUF1 * NW)

        return carry

    lax.fori_loop(0, W1 // NBUF1, outer, 0)
    for b in range(NBUF1):
        wait_store(b)

    @pl.when(wid < NT - W1 * NW)
    def _():
        t = W1 * NW + wid
        pltpu.sync_copy(tabT_hbm.at[:, pl.ds(t * LANES, LANES)], buf0)
        transposeT(0)
        pltpu.sync_copy(bufT0, lin_hbm.at[pl.ds(t * LANES * DIM, LANES * DIM)])

    @pl.when(wid == 4)
    def _():
        pltpu.sync_copy(tail_hbm, tbuf)
        pltpu.sync_copy(tbuf, lin_hbm.at[pl.ds(NT * LANES * DIM, TAIL * DIM)])


def _linearize_table(table):
    mesh = plsc.VectorSubcoreMesh(core_axis_name="c", subcore_axis_name="s")
    tabT = table.T
    tail = table[NT * LANES :].reshape(TAIL * DIM)
    f = pl.kernel(
        _lin_body,
        mesh=mesh,
        out_type=jax.ShapeDtypeStruct((V * DIM,), jnp.float32),
        scratch_types=[
            pltpu.VMEM((DIM, LANES), jnp.float32),
            pltpu.VMEM((DIM, LANES), jnp.float32),
            pltpu.VMEM((LANES * DIM,), jnp.float32),
            pltpu.VMEM((LANES * DIM,), jnp.float32),
            pltpu.VMEM((TAIL * DIM,), jnp.float32),
            pltpu.VMEM((DIM, 16), jnp.int32),
            pltpu.VMEM((DIM, 16), jnp.int32),
            pltpu.SemaphoreType.DMA((NBUF1,)),
            pltpu.SemaphoreType.DMA((NBUF1,)),
        ],
        compiler_params=pltpu.CompilerParams(
            use_tc_tiling_on_sc=True, needs_layout_passes=False
        ),
    )
    return f(tabT, tail).reshape(V, DIM)


def kernel(x_data, table):
    idx = x_data.astype(jnp.int32)
    table = _linearize_table(table)
    mesh = plsc.VectorSubcoreMesh(core_axis_name="c", subcore_axis_name="s")
    f = pl.kernel(
        _body,
        mesh=mesh,
        out_type=jax.ShapeDtypeStruct((TGRP, NBLK * SUB * LANES), jnp.float32),
        scratch_types=[
            pltpu.VMEM((NBUF, CH), jnp.int32),
            pltpu.VMEM((NBUF, CH, DIM), jnp.float32),
            pltpu.VMEM((NBUF, T4), jnp.float32),
            pltpu.VMEM((DIM, 16), jnp.int32),
            pltpu.VMEM((DIM, 16), jnp.int32),
            pltpu.SemaphoreType.DMA((NBUF,)),
            pltpu.SemaphoreType.DMA((NBUF,)),
            pltpu.SemaphoreType.DMA((NBUF,)),
        ],
        compiler_params=pltpu.CompilerParams(
            use_tc_tiling_on_sc=False, needs_layout_passes=False
        ),
    )
    o4 = f(table, idx)
    return (
        o4.reshape(TGRP, NBLK, SUB, LANES)
        .transpose(1, 3, 0, 2)
        .reshape(TOTAL, DIM)
    )


# speedup vs baseline: 6.5305x; 1.1982x over previous
import jax
import jax.numpy as jnp
from jax import lax
from jax.experimental import pallas as pl
from jax.experimental.pallas import tpu as pltpu
from jax.experimental.pallas import tpu_sc as plsc

DIM = 32
TOTAL = 1_638_400
LANES = 128
NC, NS = 2, 16
NW = NC * NS
TOK_PER_W = TOTAL // NW
G = 5
CH = G * LANES
N_CHUNKS = TOK_PER_W // CH
NBUF = 2
SUB = 8
TGRP = DIM // SUB
NBLK = TOTAL // LANES
TILE_W = G * SUB * LANES
T4 = TGRP * TILE_W


def _body(
    table_hbm, idx_hbm, o4_hbm, idx_v, rows_v, t4_v, ftab, ptab, gsems, ssems, isems
):
    wid = lax.axis_index("s") * NC + lax.axis_index("c")
    tok0 = wid * TOK_PER_W
    blk0 = wid * (TOK_PER_W // LANES)
    lane16 = lax.iota(jnp.int32, 16)

    for f in range(DIM):
        fv = lax.rem(lane16 + f, DIM)
        ftab[f, :] = fv
        ptab[f, :] = (fv // SUB) * TILE_W + lax.rem(fv, SUB) * LANES + lane16

    def prefetch_idx(b, c):
        base = tok0 + c * CH
        pltpu.async_copy(idx_hbm.at[pl.ds(base, CH)], idx_v.at[b], isems.at[b])

    def fire(b):
        pltpu.make_async_copy(
            idx_hbm.at[pl.ds(0, CH)], idx_v.at[b], isems.at[b]
        ).wait()
        for g in range(G):
            pltpu.async_copy(
                table_hbm.at[idx_v.at[b].at[pl.ds(g * LANES, LANES)]],
                rows_v.at[b].at[pl.ds(g * LANES, LANES)],
                gsems.at[b],
            )

    def drain_gathers(b):
        pltpu.make_async_copy(
            table_hbm.at[pl.ds(0, CH)], rows_v.at[b], gsems.at[b]
        ).wait()

    def wait_stores(b):
        pltpu.make_async_copy(
            o4_hbm.at[0].at[pl.ds(0, T4)], t4_v.at[b], ssems.at[b]
        ).wait()

    def transpose(b):
        @plsc.parallel_loop(0, DIM)
        def per_f(f):
            fv = ftab[f, :]
            pv = ptab[f, :]
            for g in range(G):
                for m in range(LANES // 16):
                    tok = lane16 + (g * LANES + m * 16)
                    vec = plsc.load_gather(rows_v.at[b], [tok, fv])
                    plsc.store_scatter(
                        t4_v.at[b], [pv + (g * SUB * LANES + m * 16)], vec
                    )

    def store(b, c):
        for tc in range(TGRP):
            pltpu.async_copy(
                t4_v.at[b].at[pl.ds(tc * TILE_W, TILE_W)],
                o4_hbm.at[tc].at[pl.ds((blk0 + c * G) * SUB * LANES, TILE_W)],
                ssems.at[b],
            )

    for b in range(NBUF):
        prefetch_idx(b, b)
    for b in range(NBUF):
        fire(b)

    def outer(i, carry):
        c0 = i * NBUF
        for b in range(NBUF):
            c = c0 + b
            drain_gathers(b)

            @pl.when(c + NBUF < N_CHUNKS)
            def _():
                prefetch_idx(b, c + NBUF)

            @pl.when(c >= NBUF)
            def _():
                wait_stores(b)

            transpose(b)
            store(b, c)

            @pl.when(c + NBUF < N_CHUNKS)
            def _():
                fire(b)

        return carry

    lax.fori_loop(0, N_CHUNKS // NBUF, outer, 0)
    for b in range(NBUF):
        wait_stores(b)


V = 1_000_000
NT = V // LANES
TAIL = V - NT * LANES
W1 = NT // NW
NBUF1 = 4


def _lin_body(
    tabT_hbm, tail_hbm, lin_hbm, buf0, buf1, buf2, buf3, bufT0, bufT1, bufT2,
    bufT3, tbuf, f1tab, p1tab, wsems, ssems,
):
    wid = lax.axis_index("s") * NC + lax.axis_index("c")
    lane16 = lax.iota(jnp.int32, 16)
    bufs = (buf0, buf1, buf2, buf3)
    bufTs = (bufT0, bufT1, bufT2, bufT3)

    for f in range(DIM):
        fv = lax.rem(lane16 + f, DIM)
        f1tab[f, :] = fv
        p1tab[f, :] = lane16 * DIM + fv

    def fetch(b, t):
        pltpu.async_copy(
            tabT_hbm.at[:, pl.ds(t * LANES, LANES)], bufs[b], wsems.at[b]
        )

    def wait_fetch(b):
        pltpu.make_async_copy(
            tabT_hbm.at[:, pl.ds(0, LANES)], bufs[b], wsems.at[b]
        ).wait()

    def transposeT(b):
        @plsc.parallel_loop(0, DIM)
        def per_f(f):
            fv = f1tab[f, :]
            pv = p1tab[f, :]
            for m in range(LANES // 16):
                tok = lane16 + m * 16
                vec = plsc.load_gather(bufs[b], [fv, tok])
                plsc.store_scatter(bufTs[b], [pv + m * 16 * DIM], vec)

    def store_lin(b, t):
        pltpu.async_copy(
            bufTs[b],
            lin_hbm.at[pl.ds(t * LANES * DIM, LANES * DIM)],
            ssems.at[b],
        )

    def wait_store(b):
        pltpu.make_async_copy(
            lin_hbm.at[pl.ds(0, LANES * DIM)], bufTs[b], ssems.at[b]
        ).wait()

    for b in range(NBUF1):
        fetch(b, b * NW + wid)

    def outer(j, carry):
        for b in range(NBUF1):
            t = (j * NBUF1 + b) * NW + wid
            wait_fetch(b)

            @pl.when(j >= 1)
            def _():
                wait_store(b)

            transposeT(b)
            store_lin(b, t)

            @pl.when(j < W1 // NBUF1 - 1)
            def _():
                fetch(b, t + NBUF1 * NW)

        return carry

    lax.fori_loop(0, W1 // NBUF1, outer, 0)
    for b in range(NBUF1):
        wait_store(b)

    @pl.when(wid < NT - W1 * NW)
    def _():
        t = W1 * NW + wid
        pltpu.sync_copy(tabT_hbm.at[:, pl.ds(t * LANES, LANES)], buf0)
        transposeT(0)
        pltpu.sync_copy(bufT0, lin_hbm.at[pl.ds(t * LANES * DIM, LANES * DIM)])

    @pl.when(wid == 4)
    def _():
        pltpu.sync_copy(tail_hbm, tbuf)
        pltpu.sync_copy(tbuf, lin_hbm.at[pl.ds(NT * LANES * DIM, TAIL * DIM)])


def _linearize_table(table):
    mesh = plsc.VectorSubcoreMesh(core_axis_name="c", subcore_axis_name="s")
    tabT = table.T
    tail = table[NT * LANES :].reshape(TAIL * DIM)
    f = pl.kernel(
        _lin_body,
        mesh=mesh,
        out_type=jax.ShapeDtypeStruct((V * DIM,), jnp.float32),
        scratch_types=[
            pltpu.VMEM((DIM, LANES), jnp.float32),
            pltpu.VMEM((DIM, LANES), jnp.float32),
            pltpu.VMEM((DIM, LANES), jnp.float32),
            pltpu.VMEM((DIM, LANES), jnp.float32),
            pltpu.VMEM((LANES * DIM,), jnp.float32),
            pltpu.VMEM((LANES * DIM,), jnp.float32),
            pltpu.VMEM((LANES * DIM,), jnp.float32),
            pltpu.VMEM((LANES * DIM,), jnp.float32),
            pltpu.VMEM((TAIL * DIM,), jnp.float32),
            pltpu.VMEM((DIM, 16), jnp.int32),
            pltpu.VMEM((DIM, 16), jnp.int32),
            pltpu.SemaphoreType.DMA((NBUF1,)),
            pltpu.SemaphoreType.DMA((NBUF1,)),
        ],
        compiler_params=pltpu.CompilerParams(
            use_tc_tiling_on_sc=True, needs_layout_passes=False
        ),
    )
    return f(tabT, tail).reshape(V, DIM)


def kernel(x_data, table):
    idx = x_data.astype(jnp.int32)
    table = _linearize_table(table)
    mesh = plsc.VectorSubcoreMesh(core_axis_name="c", subcore_axis_name="s")
    f = pl.kernel(
        _body,
        mesh=mesh,
        out_type=jax.ShapeDtypeStruct((TGRP, NBLK * SUB * LANES), jnp.float32),
        scratch_types=[
            pltpu.VMEM((NBUF, CH), jnp.int32),
            pltpu.VMEM((NBUF, CH, DIM), jnp.float32),
            pltpu.VMEM((NBUF, T4), jnp.float32),
            pltpu.VMEM((DIM, 16), jnp.int32),
            pltpu.VMEM((DIM, 16), jnp.int32),
            pltpu.SemaphoreType.DMA((NBUF,)),
            pltpu.SemaphoreType.DMA((NBUF,)),
            pltpu.SemaphoreType.DMA((NBUF,)),
        ],
        compiler_params=pltpu.CompilerParams(
            use_tc_tiling_on_sc=False, needs_layout_passes=False
        ),
    )
    o4 = f(table, idx)
    return (
        o4.reshape(TGRP, NBLK, SUB, LANES)
        .transpose(1, 3, 0, 2)
        .reshape(TOTAL, DIM)
    )
